# Initial kernel scaffold; baseline (speedup 1.0000x reference)
import jax
import jax.numpy as jnp
from jax.experimental import pallas as pl


def kernel(token_ids, weights):
    raise NotImplementedError("write your pallas kernel here")


# speedup vs baseline: 8.1493x; 8.1493x over previous
import functools

import jax
import jax.numpy as jnp
from jax import lax
from jax.experimental import pallas as pl
from jax.experimental.pallas import tpu as pltpu
from jax.experimental.pallas import tpu_sc as plsc

_INFO = plsc.get_sparse_core_info()
_NC = _INFO.num_cores
_NS = _INFO.num_subcores
_NW = _NC * _NS
_IPG = 128


def _make_gather(V, D, B, chunk_rows):
  assert B % (_NW * chunk_rows * _IPG) == 0
  rows_per_w = B // _NW
  chunk = chunk_rows * _IPG
  n_iter = rows_per_w // chunk
  mesh = plsc.VectorSubcoreMesh(core_axis_name="c", subcore_axis_name="s")

  @functools.partial(
      pl.kernel,
      mesh=mesh,
      out_type=jax.ShapeDtypeStruct((B, D), jnp.float32),
      scratch_types=[
          pltpu.VMEM((chunk_rows, _IPG), jnp.int32),
          pltpu.VMEM((chunk, D), jnp.float32),
          pltpu.SemaphoreType.DMA,
      ],
  )
  def k(table_hbm, idx_hbm, out_hbm, idx_v, rows_v, sem):
    wid = lax.axis_index("s") * _NC + lax.axis_index("c")
    row0 = wid * (rows_per_w // _IPG)

    def body(i, carry):
      base_row = row0 + i * chunk_rows
      pltpu.sync_copy(idx_hbm.at[pl.ds(base_row, chunk_rows)], idx_v)
      copies = []
      for j in range(chunk_rows):
        copies.append(
            pltpu.async_copy(
                table_hbm.at[idx_v.at[j]],
                rows_v.at[pl.ds(j * _IPG, _IPG)],
                sem,
            ))
      for c in copies:
        c.wait()
      pltpu.sync_copy(rows_v, out_hbm.at[pl.ds(base_row * _IPG, chunk)])
      return carry

    lax.fori_loop(0, n_iter, body, 0)

  return k


def kernel(token_ids, weights):
  B0, B1 = token_ids.shape
  V, D = weights.shape
  B = B0 * B1
  idx = token_ids.reshape(B // _IPG, _IPG).astype(jnp.int32)
  out = _make_gather(V, D, B, chunk_rows=4)(weights, idx)
  return out.reshape(B0, B1, D)


# speedup vs baseline: 9.1546x; 1.1234x over previous
import functools

import jax
import jax.numpy as jnp
from jax import lax
from jax.experimental import pallas as pl
from jax.experimental.pallas import tpu as pltpu
from jax.experimental.pallas import tpu_sc as plsc

_INFO = plsc.get_sparse_core_info()
_NC = _INFO.num_cores
_NS = _INFO.num_subcores
_NW = _NC * _NS
_IPG = 128


def _make_gather(V, D, B, chunk_rows):
  rows_per_w = B // _NW
  chunk = chunk_rows * _IPG
  n_chunks = rows_per_w // chunk
  assert rows_per_w % chunk == 0 and n_chunks % 2 == 0
  mesh = plsc.VectorSubcoreMesh(core_axis_name="c", subcore_axis_name="s")

  @functools.partial(
      pl.kernel,
      mesh=mesh,
      out_type=jax.ShapeDtypeStruct((B, D), jnp.float32),
      scratch_types=[
          pltpu.VMEM((chunk_rows, _IPG), jnp.int32),
          pltpu.VMEM((chunk_rows, _IPG), jnp.int32),
          pltpu.VMEM((chunk, D), jnp.float32),
          pltpu.VMEM((chunk, D), jnp.float32),
          pltpu.SemaphoreType.DMA,
          pltpu.SemaphoreType.DMA,
          pltpu.SemaphoreType.DMA,
          pltpu.SemaphoreType.DMA,
          pltpu.SemaphoreType.DMA,
          pltpu.SemaphoreType.DMA,
      ],
  )
  def k(table_hbm, idx_hbm, out_hbm, idx_a, idx_b, rows_a, rows_b,
        s_ia, s_ib, s_ga, s_gb, s_wa, s_wb):
    wid = lax.axis_index("s") * _NC + lax.axis_index("c")
    row0 = wid * (rows_per_w // _IPG)

    def idx_rows(i):
      return pl.ds(row0 + i * chunk_rows, chunk_rows)

    def out_rows(i):
      return pl.ds((row0 + i * chunk_rows) * _IPG, chunk)

    pltpu.async_copy(idx_hbm.at[idx_rows(0)], idx_a, s_ia)
    pltpu.async_copy(idx_hbm.at[idx_rows(1)], idx_b, s_ib)

    def process(g, i, idx_v, rows_v, s_i, s_g, s_w):
      pltpu.make_async_copy(idx_hbm.at[idx_rows(0)], idx_v, s_i).wait()

      @pl.when(g > 0)
      def _():
        pltpu.make_async_copy(rows_v, out_hbm.at[out_rows(0)], s_w).wait()

      copies = [
          pltpu.async_copy(table_hbm.at[idx_v.at[j]],
                           rows_v.at[pl.ds(j * _IPG, _IPG)], s_g)
          for j in range(chunk_rows)
      ]
      for c in copies:
        c.wait()
      pltpu.async_copy(rows_v, out_hbm.at[out_rows(i)], s_w)
      nxt = jnp.minimum(i + 2, n_chunks - 1)
      pltpu.async_copy(idx_hbm.at[idx_rows(nxt)], idx_v, s_i)

    def body(g, carry):
      process(g, 2 * g, idx_a, rows_a, s_ia, s_ga, s_wa)
      process(g, 2 * g + 1, idx_b, rows_b, s_ib, s_gb, s_wb)
      return carry

    lax.fori_loop(0, n_chunks // 2, body, 0)

    pltpu.make_async_copy(idx_hbm.at[idx_rows(0)], idx_a, s_ia).wait()
    pltpu.make_async_copy(idx_hbm.at[idx_rows(0)], idx_b, s_ib).wait()
    pltpu.make_async_copy(rows_a, out_hbm.at[out_rows(0)], s_wa).wait()
    pltpu.make_async_copy(rows_b, out_hbm.at[out_rows(0)], s_wb).wait()

  return k


def kernel(token_ids, weights):
  B0, B1 = token_ids.shape
  V, D = weights.shape
  B = B0 * B1
  idx = token_ids.reshape(B // _IPG, _IPG).astype(jnp.int32)
  out = _make_gather(V, D, B, chunk_rows=2)(weights, idx)
  return out.reshape(B0, B1, D)


# speedup vs baseline: 9.1672x; 1.0014x over previous
import functools

import jax
import jax.numpy as jnp
from jax import lax
from jax.experimental import pallas as pl
from jax.experimental.pallas import tpu as pltpu
from jax.experimental.pallas import tpu_sc as plsc

_INFO = plsc.get_sparse_core_info()
_NC = _INFO.num_cores
_NS = _INFO.num_subcores
_NW = _NC * _NS
_IPG = 128
_RING = 5
_LAG = 2


def _make_gather(V, D, B):
  n = B // _NW // _IPG
  assert B % (_NW * _IPG) == 0 and n % _RING == 0 and n >= _RING
  mesh = plsc.VectorSubcoreMesh(core_axis_name="c", subcore_axis_name="s")

  @functools.partial(
      pl.kernel,
      mesh=mesh,
      out_type=jax.ShapeDtypeStruct((B, D), jnp.float32),
      scratch_types=(
          [pltpu.VMEM((n, _IPG), jnp.int32)]
          + [pltpu.VMEM((_IPG, D), jnp.float32)] * _RING
          + [pltpu.SemaphoreType.DMA] * (2 * _RING)
      ),
  )
  def k(table_hbm, idx_hbm, out_hbm, idx_all, *bufs_and_sems):
    rows = bufs_and_sems[:_RING]
    s_g = bufs_and_sems[_RING:2 * _RING]
    s_w = bufs_and_sems[2 * _RING:]
    wid = lax.axis_index("s") * _NC + lax.axis_index("c")
    row0 = wid * n

    pltpu.sync_copy(idx_hbm.at[pl.ds(row0, n)], idx_all)

    def out_chunk(j):
      return out_hbm.at[pl.ds((row0 + j) * _IPG, _IPG)]

    def fire(j, p):
      pltpu.async_copy(table_hbm.at[idx_all.at[j]], rows[p], s_g[p])

    def drain(j, p):
      pltpu.make_async_copy(table_hbm.at[idx_all.at[0]], rows[p],
                            s_g[p]).wait()
      pltpu.async_copy(rows[p], out_chunk(j), s_w[p])

    for r in range(_RING):
      fire(r, r)
      if r >= _LAG:
        drain(r - _LAG, r - _LAG)

    def body(g, carry):
      for r in range(_RING):
        i = g * _RING + r
        pltpu.make_async_copy(rows[r], out_chunk(0), s_w[r]).wait()
        fire(i, r)
        drain(i - _LAG, (r - _LAG) % _RING)
      return carry

    lax.fori_loop(1, n // _RING, body, 0)

    for j in range(n - _LAG, n):
      p = j % _RING
      pltpu.make_async_copy(table_hbm.at[idx_all.at[0]], rows[p],
                            s_g[p]).wait()
      pltpu.async_copy(rows[p], out_hbm.at[pl.ds((row0 + j) * _IPG, _IPG)],
                       s_w[p])
    for r in range(_RING):
      pltpu.make_async_copy(rows[r], out_chunk(0), s_w[r]).wait()

  return k


def kernel(token_ids, weights):
  B0, B1 = token_ids.shape
  V, D = weights.shape
  B = B0 * B1
  idx = token_ids.reshape(B // _IPG, _IPG).astype(jnp.int32)
  out = _make_gather(V, D, B)(weights, idx)
  return out.reshape(B0, B1, D)
